# baseline (speedup 1.0000x reference)
import functools

import jax
import jax.numpy as jnp
from jax.experimental import pallas as pl
from jax.experimental.pallas import tpu as pltpu

K = 2048
BJ = 512
RCHUNK = 256


def _perm_kernel(x_ref, out_ref, rank_ref):
    jstep = pl.program_id(1)

    @pl.when(jstep == 0)
    def _compute_rank():
        x = x_ref[0, 0, :]
        col = x[None, :]
        i_idx = jax.lax.broadcasted_iota(jnp.int32, (1, K), 1)

        rank = jnp.zeros((1, K), jnp.float32)
        for c in range(K // RCHUNK):
            xc = jnp.reshape(x[c * RCHUNK:(c + 1) * RCHUNK],
                             (RCHUNK, 1))
            j_idx = (jax.lax.broadcasted_iota(jnp.int32, (RCHUNK, 1), 0)
                     + c * RCHUNK)
            gt = xc > col
            tie = (xc == col) & (j_idx < i_idx)
            rank = rank + jnp.sum((gt | tie).astype(jnp.float32),
                                  axis=0, keepdims=True)
        rank_ref[...] = rank

    j0 = jstep * BJ
    row = (jax.lax.broadcasted_iota(jnp.int32, (BJ, K), 0) + j0)
    rank = rank_ref[0, :][None, :]
    out_ref[0, :, :] = (row.astype(jnp.float32) == rank).astype(jnp.float32)


@jax.jit
def kernel(input):
    N, k = input.shape
    assert k == K
    grid = (N, K // BJ)
    x3 = input.reshape(N, 1, K)
    return pl.pallas_call(
        _perm_kernel,
        grid=grid,
        in_specs=[pl.BlockSpec((1, 1, K), lambda n, j: (n, 0, 0))],
        out_specs=pl.BlockSpec((1, BJ, K), lambda n, j: (n, j, 0)),
        out_shape=jax.ShapeDtypeStruct((N, K, K), input.dtype),
        scratch_shapes=[pltpu.VMEM((1, K), jnp.float32)],
        compiler_params=pltpu.CompilerParams(
            dimension_semantics=("arbitrary", "arbitrary"),
        ),
    )(x3)
